# Initial kernel scaffold; baseline (speedup 1.0000x reference)
import jax
import jax.numpy as jnp
from jax.experimental import pallas as pl


def kernel(raw_preds, vid_features, reg_features):
    raise NotImplementedError("write your pallas kernel here")


# speedup vs baseline: 12.7002x; 12.7002x over previous
import jax
import jax.numpy as jnp
from jax import lax
from jax.experimental import pallas as pl
from jax.experimental.pallas import tpu as pltpu
from jax.experimental.pallas import tpu_sc as plsc

_A = 20000
_R, _C = 8, 2500
_K = 750
_KEEP = 30
_NCLS = 30
_IOU = 0.75
_SLOTS = 32
_D = 256

_GB = 256
_NW = 32
_BPW = _GB // _NW


def _select_body(raw_ref, idx_ref, box_ref, score_ref):
    x1 = raw_ref[0, 0]
    y1 = raw_ref[0, 1]
    x2 = raw_ref[0, 2]
    y2 = raw_ref[0, 3]

    s = raw_ref[0, 4]
    c = jnp.zeros_like(s)
    for j in range(1, _NCLS):
        v = raw_ref[0, 4 + j]
        c = jnp.where(v > s, jnp.float32(j), c)
        s = jnp.maximum(s, v)

    bits = lax.bitcast_convert_type(s, jnp.int32)
    ridx = lax.broadcasted_iota(jnp.int32, (_R, _C), 0)
    cidx = lax.broadcasted_iota(jnp.int32, (_R, _C), 1)
    idx = ridx * _C + cidx

    def bs_val(_, lohi):
        lo, hi = lohi
        mid = lo + (hi - lo) // 2
        ge = jnp.sum((bits > mid).astype(jnp.int32)) >= _K
        return (jnp.where(ge, mid, lo), jnp.where(ge, hi, mid))

    _, t = lax.fori_loop(0, 31, bs_val, (jnp.int32(-1), jnp.int32(0x7F800000)))

    need = _K - jnp.sum((bits > t).astype(jnp.int32))
    eq = bits == t

    def bs_idx(_, lohi):
        lo, hi = lohi
        mid = lo + (hi - lo) // 2
        ge = jnp.sum((eq & (idx < mid)).astype(jnp.int32)) >= need
        return (jnp.where(ge, lo, mid), jnp.where(ge, mid, hi))

    _, cut = lax.fori_loop(0, 15, bs_idx, (jnp.int32(0), jnp.int32(_A)))

    avail = (bits > t) | (eq & (idx < cut))

    neg = jnp.float32(-jnp.inf)
    bigi = jnp.int32(2**30)

    m4 = jnp.maximum(jnp.maximum(x1, y1), jnp.maximum(x2, y2))
    max_coord = jnp.max(jnp.where(avail, m4, neg))
    shift = max_coord + 1.0
    off = c * shift
    ox1 = x1 + off
    oy1 = y1 + off
    ox2 = x2 + off
    oy2 = y2 + off
    area = (x2 - x1) * (y2 - y1)

    def extract(vec, mask_p):
        return jnp.sum(jnp.where(mask_p, vec, 0.0))

    s_av = jnp.where(avail, s, neg)
    top_m = jnp.max(s_av)
    top_p = jnp.min(jnp.where(avail & (s == top_m), idx, bigi))

    slot = lax.broadcasted_iota(jnp.int32, (1, _SLOTS), 1)
    idx_acc = jnp.zeros((1, _SLOTS), jnp.int32)
    sc_acc = jnp.zeros((1, _SLOTS), jnp.float32)
    bx = [jnp.zeros((1, _SLOTS), jnp.float32) for _ in range(4)]

    avail_cur = avail
    for i in range(_KEEP):
        sa = jnp.where(avail_cur, s, neg)
        m = jnp.max(sa)
        valid = m > neg
        p0 = jnp.min(jnp.where(avail_cur & (s == m), idx, bigi))
        p = jnp.where(valid, p0, top_p)
        mask_p = idx == p
        px1 = extract(x1, mask_p)
        py1 = extract(y1, mask_p)
        px2 = extract(x2, mask_p)
        py2 = extract(y2, mask_p)
        pc = extract(c, mask_p)
        ps = jnp.where(valid, m, top_m)
        po = pc * shift
        ix1 = jnp.maximum(px1 + po, ox1)
        iy1 = jnp.maximum(py1 + po, oy1)
        ix2 = jnp.minimum(px2 + po, ox2)
        iy2 = jnp.minimum(py2 + po, oy2)
        inter = jnp.maximum(ix2 - ix1, 0.0) * jnp.maximum(iy2 - iy1, 0.0)
        area_r = (px2 - px1) * (py2 - py1)
        iou = inter / (area_r + area - inter + 1e-9)
        sup = ((iou > _IOU) | mask_p) & valid
        avail_cur = avail_cur & jnp.logical_not(sup)
        hit = slot == i
        idx_acc = jnp.where(hit, p, idx_acc)
        sc_acc = jnp.where(hit, ps, sc_acc)
        bx[0] = jnp.where(hit, px1, bx[0])
        bx[1] = jnp.where(hit, py1, bx[1])
        bx[2] = jnp.where(hit, px2, bx[2])
        bx[3] = jnp.where(hit, py2, bx[3])

    idx_ref[0] = idx_acc
    score_ref[0] = sc_acc
    box_ref[0] = jnp.concatenate(bx, axis=0)


def _gather_body(vid_hbm, reg_hbm, idx_hbm, out_v, out_r,
                 idx_v, rows_v, rows_r, sem_v, sem_r):
    wid = lax.axis_index("s") * 2 + lax.axis_index("c")
    base = wid * _BPW
    pltpu.sync_copy(idx_hbm.at[pl.ds(base, _BPW)], idx_v)
    cp_v = pltpu.async_copy(vid_hbm.at[idx_v], rows_v, sem_v)
    cp_r = pltpu.async_copy(reg_hbm.at[idx_v], rows_r, sem_r)
    cp_v.wait()
    cp_r.wait()
    pltpu.sync_copy(rows_v, out_v.at[pl.ds(base, _BPW)])
    pltpu.sync_copy(rows_r, out_r.at[pl.ds(base, _BPW)])


def _sc_gather(vid_flat, reg_flat, gidx):
    mesh = plsc.VectorSubcoreMesh(core_axis_name="c", subcore_axis_name="s")
    k = pl.kernel(
        _gather_body,
        mesh=mesh,
        out_type=[
            jax.ShapeDtypeStruct((_GB, _D), jnp.float32),
            jax.ShapeDtypeStruct((_GB, _D), jnp.float32),
        ],
        scratch_types=[
            pltpu.VMEM((_BPW,), jnp.int32),
            pltpu.VMEM((_BPW, _D), jnp.float32),
            pltpu.VMEM((_BPW, _D), jnp.float32),
            pltpu.SemaphoreType.DMA,
            pltpu.SemaphoreType.DMA,
        ],
    )
    return k(vid_flat, reg_flat, gidx)


def kernel(raw_preds, vid_features, reg_features):
    B = raw_preds.shape[0]
    raw4 = raw_preds.reshape(B, 4 + _NCLS, _R, _C)
    idx_out, box_out, score_out = pl.pallas_call(
        _select_body,
        grid=(B,),
        in_specs=[pl.BlockSpec((1, 4 + _NCLS, _R, _C), lambda b: (b, 0, 0, 0))],
        out_specs=[
            pl.BlockSpec((1, 1, _SLOTS), lambda b: (b, 0, 0)),
            pl.BlockSpec((1, 4, _SLOTS), lambda b: (b, 0, 0)),
            pl.BlockSpec((1, 1, _SLOTS), lambda b: (b, 0, 0)),
        ],
        out_shape=[
            jax.ShapeDtypeStruct((B, 1, _SLOTS), jnp.int32),
            jax.ShapeDtypeStruct((B, 4, _SLOTS), jnp.float32),
            jax.ShapeDtypeStruct((B, 1, _SLOTS), jnp.float32),
        ],
    )(raw4)

    keep_idx = idx_out[:, 0, :_KEEP]
    boxes = jnp.transpose(box_out[:, :, :_KEEP], (0, 2, 1)).reshape(1, B * _KEEP, 4)
    scores = score_out[:, 0, :_KEEP].reshape(1, B * _KEEP)

    gidx = (keep_idx + jnp.arange(B, dtype=jnp.int32)[:, None] * _A).reshape(-1)
    gidx_pad = jnp.zeros((_GB,), jnp.int32).at[: B * _KEEP].set(gidx)
    vid_flat = vid_features.reshape(B * _A, _D)
    reg_flat = reg_features.reshape(B * _A, _D)
    vrows, rrows = _sc_gather(vid_flat, reg_flat, gidx_pad)
    concat_cls = vrows[: B * _KEEP].reshape(1, B * _KEEP, _D)
    concat_reg = rrows[: B * _KEEP].reshape(1, B * _KEEP, _D)
    return concat_cls, concat_reg, boxes, scores


# speedup vs baseline: 21.4635x; 1.6900x over previous
import jax
import jax.numpy as jnp
from jax import lax
from jax.experimental import pallas as pl
from jax.experimental.pallas import tpu as pltpu
from jax.experimental.pallas import tpu_sc as plsc

_B = 4
_A = 20000
_R, _C = 8, 2500
_K = 750
_KEEP = 30
_NCLS = 30
_IOU = 0.75
_SLOTS = 64
_D = 256

_GB = _B * _SLOTS
_NW = 32
_BPW = _GB // _NW


def _red_max(x):
    return jnp.max(x, axis=(1, 2))


def _red_min(x):
    return jnp.min(x, axis=(1, 2))


def _red_sum_bool(x):
    return jnp.sum(x.astype(jnp.int32), axis=(1, 2))


def _select_body(raw_ref, gidx_ref, box_ref, score_ref):
    x1 = raw_ref[:, 0]
    y1 = raw_ref[:, 1]
    x2 = raw_ref[:, 2]
    y2 = raw_ref[:, 3]

    s = raw_ref[:, 4]
    c = jnp.zeros_like(s)
    for j in range(1, _NCLS):
        v = raw_ref[:, 4 + j]
        c = jnp.where(v > s, jnp.float32(j), c)
        s = jnp.maximum(s, v)

    bits = lax.bitcast_convert_type(s, jnp.int32)
    ridx = lax.broadcasted_iota(jnp.int32, (_B, _R, _C), 1)
    cidx = lax.broadcasted_iota(jnp.int32, (_B, _R, _C), 2)
    idx = ridx * _C + cidx

    kk = jnp.int32(_K)

    def val_probe(lohi, nprobe):
        lo, hi = lohi
        step = (hi - lo) >> 3 if nprobe == 7 else (hi - lo) >> 1
        ms = [lo + step * (j + 1) for j in range(nprobe)]
        ges = [_red_sum_bool(bits > m[:, None, None]) >= kk for m in ms]
        for j in range(nprobe):
            lo = jnp.where(ges[j], ms[j], lo)
        for j in reversed(range(nprobe)):
            hi = jnp.where(ges[j], hi, ms[j])
        return lo, hi

    lohi = (jnp.full((_B,), -1, jnp.int32), jnp.full((_B,), 0x7F800000, jnp.int32))
    lohi = lax.fori_loop(0, 10, lambda _, lh: val_probe(lh, 7), lohi)
    lohi = lax.fori_loop(0, 4, lambda _, lh: val_probe(lh, 1), lohi)
    t = lohi[1]
    t3 = t[:, None, None]

    need = kk - _red_sum_bool(bits > t3)
    eq = bits == t3

    def idx_probe(lohi, nprobe):
        lo, hi = lohi
        step = (hi - lo) >> 3 if nprobe == 7 else (hi - lo) >> 1
        ms = [lo + step * (j + 1) for j in range(nprobe)]
        ges = [_red_sum_bool(eq & (idx < m[:, None, None])) >= need for m in ms]
        for j in range(nprobe):
            hi = jnp.where(ges[j], jnp.minimum(hi, ms[j]), hi)
        for j in reversed(range(nprobe)):
            lo = jnp.where(ges[j], lo, jnp.maximum(lo, ms[j]))
        return lo, hi

    lohi = (jnp.zeros((_B,), jnp.int32), jnp.full((_B,), _A, jnp.int32))
    lohi = lax.fori_loop(0, 4, lambda _, lh: idx_probe(lh, 7), lohi)
    lohi = lax.fori_loop(0, 4, lambda _, lh: idx_probe(lh, 1), lohi)
    cut = lohi[1]

    avail = (bits > t3) | (eq & (idx < cut[:, None, None]))

    neg = jnp.float32(-jnp.inf)
    bigi = jnp.int32(2**30)

    m4 = jnp.maximum(jnp.maximum(x1, y1), jnp.maximum(x2, y2))
    max_coord = _red_max(jnp.where(avail, m4, neg))
    shift = max_coord + 1.0
    off = c * shift[:, None, None]
    ox1 = x1 + off
    oy1 = y1 + off
    ox2 = x2 + off
    oy2 = y2 + off
    area = (x2 - x1) * (y2 - y1)

    def extract(vec, mask_p):
        return jnp.sum(jnp.where(mask_p, vec, 0.0), axis=(1, 2))

    s_av = jnp.where(avail, s, neg)
    top_m = _red_max(s_av)
    top_p = _red_min(jnp.where(avail & (s == top_m[:, None, None]), idx, bigi))

    slot = lax.broadcasted_iota(jnp.int32, (_B, _SLOTS), 1)
    boff = lax.broadcasted_iota(jnp.int32, (_B, _SLOTS), 0) * _A
    gidx_acc = jnp.zeros((_B, _SLOTS), jnp.int32)
    sc_acc = jnp.zeros((_B, _SLOTS), jnp.float32)
    bx = [jnp.zeros((_B, _SLOTS), jnp.float32) for _ in range(4)]

    avail_cur = avail
    for i in range(_KEEP):
        sa = jnp.where(avail_cur, s, neg)
        m = _red_max(sa)
        valid = m > neg
        p0 = _red_min(jnp.where(avail_cur & (s == m[:, None, None]), idx, bigi))
        p = jnp.where(valid, p0, top_p)
        mask_p = idx == p[:, None, None]
        px1 = extract(x1, mask_p)
        py1 = extract(y1, mask_p)
        px2 = extract(x2, mask_p)
        py2 = extract(y2, mask_p)
        pc = extract(c, mask_p)
        ps = jnp.where(valid, m, top_m)
        po = pc * shift
        ix1 = jnp.maximum((px1 + po)[:, None, None], ox1)
        iy1 = jnp.maximum((py1 + po)[:, None, None], oy1)
        ix2 = jnp.minimum((px2 + po)[:, None, None], ox2)
        iy2 = jnp.minimum((py2 + po)[:, None, None], oy2)
        inter = jnp.maximum(ix2 - ix1, 0.0) * jnp.maximum(iy2 - iy1, 0.0)
        area_r = (px2 - px1) * (py2 - py1)
        iou = inter / ((area_r[:, None, None] + area - inter) + 1e-9)
        sup = ((iou > _IOU) | mask_p) & valid[:, None, None]
        avail_cur = avail_cur & jnp.logical_not(sup)
        hit = slot == i
        gidx_acc = jnp.where(hit, p[:, None] + boff, gidx_acc)
        sc_acc = jnp.where(hit, ps[:, None], sc_acc)
        bx[0] = jnp.where(hit, px1[:, None], bx[0])
        bx[1] = jnp.where(hit, py1[:, None], bx[1])
        bx[2] = jnp.where(hit, px2[:, None], bx[2])
        bx[3] = jnp.where(hit, py2[:, None], bx[3])

    gidx_ref[...] = gidx_acc
    score_ref[...] = sc_acc
    for j in range(4):
        box_ref[:, j, :] = bx[j]


def _gather_body(vid_hbm, reg_hbm, idx_hbm, out_v, out_r,
                 idx_v, rows_v, rows_r, sem_v, sem_r):
    wid = lax.axis_index("s") * 2 + lax.axis_index("c")
    base = wid * _BPW
    pltpu.sync_copy(idx_hbm.at[pl.ds(base, _BPW)], idx_v)
    cp_v = pltpu.async_copy(vid_hbm.at[idx_v], rows_v, sem_v)
    cp_r = pltpu.async_copy(reg_hbm.at[idx_v], rows_r, sem_r)
    cp_v.wait()
    cp_r.wait()
    pltpu.sync_copy(rows_v, out_v.at[pl.ds(base, _BPW)])
    pltpu.sync_copy(rows_r, out_r.at[pl.ds(base, _BPW)])


def _sc_gather(vid_flat, reg_flat, gidx):
    mesh = plsc.VectorSubcoreMesh(core_axis_name="c", subcore_axis_name="s")
    k = pl.kernel(
        _gather_body,
        mesh=mesh,
        out_type=[
            jax.ShapeDtypeStruct((_GB, _D), jnp.float32),
            jax.ShapeDtypeStruct((_GB, _D), jnp.float32),
        ],
        scratch_types=[
            pltpu.VMEM((_BPW,), jnp.int32),
            pltpu.VMEM((_BPW, _D), jnp.float32),
            pltpu.VMEM((_BPW, _D), jnp.float32),
            pltpu.SemaphoreType.DMA,
            pltpu.SemaphoreType.DMA,
        ],
    )
    return k(vid_flat, reg_flat, gidx)


def kernel(raw_preds, vid_features, reg_features):
    raw4 = raw_preds.reshape(_B, 4 + _NCLS, _R, _C)
    gidx_out, box_out, score_out = pl.pallas_call(
        _select_body,
        out_shape=[
            jax.ShapeDtypeStruct((_B, _SLOTS), jnp.int32),
            jax.ShapeDtypeStruct((_B, 4, _SLOTS), jnp.float32),
            jax.ShapeDtypeStruct((_B, _SLOTS), jnp.float32),
        ],
    )(raw4)

    boxes = jnp.transpose(box_out[:, :, :_KEEP], (0, 2, 1)).reshape(1, _B * _KEEP, 4)
    scores = score_out[:, :_KEEP].reshape(1, _B * _KEEP)

    vid_flat = vid_features.reshape(_B * _A, _D)
    reg_flat = reg_features.reshape(_B * _A, _D)
    vrows, rrows = _sc_gather(vid_flat, reg_flat, gidx_out.reshape(_GB))
    concat_cls = vrows.reshape(_B, _SLOTS, _D)[:, :_KEEP].reshape(1, _B * _KEEP, _D)
    concat_reg = rrows.reshape(_B, _SLOTS, _D)[:, :_KEEP].reshape(1, _B * _KEEP, _D)
    return concat_cls, concat_reg, boxes, scores


# speedup vs baseline: 22.4597x; 1.0464x over previous
import jax
import jax.numpy as jnp
from jax import lax
from jax.experimental import pallas as pl
from jax.experimental.pallas import tpu as pltpu
from jax.experimental.pallas import tpu_sc as plsc

_B = 4
_A = 20000
_R, _C = 8, 2500
_K = 750
_KEEP = 30
_NCLS = 30
_IOU = 0.75
_SLOTS = 64
_D = 256

_GB = _B * _SLOTS
_NW = 32
_BPW = _GB // _NW


def _red_max(x):
    return jnp.max(x, axis=(1, 2))


def _red_min(x):
    return jnp.min(x, axis=(1, 2))


def _red_sum_bool(x):
    return jnp.sum(x.astype(jnp.int32), axis=(1, 2))


def _select_body(raw_ref, gidx_ref, box_ref, score_ref):
    x1 = raw_ref[:, 0]
    y1 = raw_ref[:, 1]
    x2 = raw_ref[:, 2]
    y2 = raw_ref[:, 3]

    s = raw_ref[:, 4]
    c = jnp.zeros_like(s)
    for j in range(1, _NCLS):
        v = raw_ref[:, 4 + j]
        c = jnp.where(v > s, jnp.float32(j), c)
        s = jnp.maximum(s, v)

    bits = lax.bitcast_convert_type(s, jnp.int32)
    ridx = lax.broadcasted_iota(jnp.int32, (_B, _R, _C), 1)
    cidx = lax.broadcasted_iota(jnp.int32, (_B, _R, _C), 2)
    idx = ridx * _C + cidx

    kk = jnp.int32(_K)

    def val_probe(lohi, nprobe):
        lo, hi = lohi
        step = (hi - lo) >> 3 if nprobe == 7 else (hi - lo) >> 1
        ms = [lo + step * (j + 1) for j in range(nprobe)]
        ges = [_red_sum_bool(bits > m[:, None, None]) >= kk for m in ms]
        for j in range(nprobe):
            lo = jnp.where(ges[j], ms[j], lo)
        for j in reversed(range(nprobe)):
            hi = jnp.where(ges[j], hi, ms[j])
        return lo, hi

    lohi = (jnp.full((_B,), -1, jnp.int32), jnp.full((_B,), 0x7F800000, jnp.int32))
    lohi = lax.fori_loop(0, 10, lambda _, lh: val_probe(lh, 7), lohi)
    lohi = lax.fori_loop(0, 4, lambda _, lh: val_probe(lh, 1), lohi)
    t = lohi[1]
    t3 = t[:, None, None]

    need = kk - _red_sum_bool(bits > t3)
    eq = bits == t3

    def idx_probe(lohi, nprobe):
        lo, hi = lohi
        step = (hi - lo) >> 3 if nprobe == 7 else (hi - lo) >> 1
        ms = [lo + step * (j + 1) for j in range(nprobe)]
        ges = [_red_sum_bool(eq & (idx < m[:, None, None])) >= need for m in ms]
        for j in range(nprobe):
            hi = jnp.where(ges[j], jnp.minimum(hi, ms[j]), hi)
        for j in reversed(range(nprobe)):
            lo = jnp.where(ges[j], lo, jnp.maximum(lo, ms[j]))
        return lo, hi

    lohi = (jnp.zeros((_B,), jnp.int32), jnp.full((_B,), _A, jnp.int32))
    lohi = lax.fori_loop(0, 4, lambda _, lh: idx_probe(lh, 7), lohi)
    lohi = lax.fori_loop(0, 4, lambda _, lh: idx_probe(lh, 1), lohi)
    cut = lohi[1]

    avail = (bits > t3) | (eq & (idx < cut[:, None, None]))

    neg = jnp.float32(-jnp.inf)
    bigi = jnp.int32(2**30)

    m4 = jnp.maximum(jnp.maximum(x1, y1), jnp.maximum(x2, y2))
    max_coord = _red_max(jnp.where(avail, m4, neg))
    shift = max_coord + 1.0
    off = c * shift[:, None, None]
    ox1 = x1 + off
    oy1 = y1 + off
    ox2 = x2 + off
    oy2 = y2 + off
    area = (x2 - x1) * (y2 - y1)

    def extract(vec, mask_p):
        return jnp.sum(jnp.where(mask_p, vec, 0.0), axis=(1, 2))

    s_av = jnp.where(avail, s, neg)
    top_m = _red_max(s_av)
    top_p = _red_min(jnp.where(avail & (s == top_m[:, None, None]), idx, bigi))

    slot = lax.broadcasted_iota(jnp.int32, (_B, _SLOTS), 1)
    gidx_acc = jnp.zeros((_B, _SLOTS), jnp.int32)
    sc_acc = jnp.zeros((_B, _SLOTS), jnp.float32)
    bx = [jnp.zeros((_B, _SLOTS), jnp.float32) for _ in range(4)]

    avail_cur = avail
    for i in range(_KEEP):
        sa = jnp.where(avail_cur, s, neg)
        m = _red_max(sa)
        valid = m > neg
        p0 = _red_min(jnp.where(avail_cur & (s == m[:, None, None]), idx, bigi))
        p = jnp.where(valid, p0, top_p)
        mask_p = idx == p[:, None, None]
        px1 = extract(x1, mask_p)
        py1 = extract(y1, mask_p)
        px2 = extract(x2, mask_p)
        py2 = extract(y2, mask_p)
        pc = extract(c, mask_p)
        ps = jnp.where(valid, m, top_m)
        po = pc * shift
        ix1 = jnp.maximum((px1 + po)[:, None, None], ox1)
        iy1 = jnp.maximum((py1 + po)[:, None, None], oy1)
        ix2 = jnp.minimum((px2 + po)[:, None, None], ox2)
        iy2 = jnp.minimum((py2 + po)[:, None, None], oy2)
        inter = jnp.maximum(ix2 - ix1, 0.0) * jnp.maximum(iy2 - iy1, 0.0)
        area_r = (px2 - px1) * (py2 - py1)
        iou = inter / ((area_r[:, None, None] + area - inter) + 1e-9)
        sup = ((iou > _IOU) | mask_p) & valid[:, None, None]
        avail_cur = avail_cur & jnp.logical_not(sup)
        hit = slot == i
        gidx_acc = jnp.where(hit, p[:, None], gidx_acc)
        sc_acc = jnp.where(hit, ps[:, None], sc_acc)
        bx[0] = jnp.where(hit, px1[:, None], bx[0])
        bx[1] = jnp.where(hit, py1[:, None], bx[1])
        bx[2] = jnp.where(hit, px2[:, None], bx[2])
        bx[3] = jnp.where(hit, py2[:, None], bx[3])

    gidx_ref[...] = gidx_acc
    score_ref[...] = sc_acc
    for j in range(4):
        box_ref[:, j, :] = bx[j]


def _gather_body(vid_hbm, reg_hbm, idx_hbm, out_v, out_r,
                 idx_v, rows_v, rows_r, sem_v, sem_r):
    wid = lax.axis_index("s") * 2 + lax.axis_index("c")
    base = wid * _BPW
    b = base // _SLOTS
    pltpu.sync_copy(idx_hbm.at[pl.ds(base, _BPW)], idx_v)
    cp_v = pltpu.async_copy(vid_hbm.at[b].at[idx_v], rows_v, sem_v)
    cp_r = pltpu.async_copy(reg_hbm.at[b].at[idx_v], rows_r, sem_r)
    cp_v.wait()
    cp_r.wait()
    pltpu.sync_copy(rows_v, out_v.at[pl.ds(base, _BPW)])
    pltpu.sync_copy(rows_r, out_r.at[pl.ds(base, _BPW)])


def _sc_gather(vid_feats, reg_feats, gidx):
    mesh = plsc.VectorSubcoreMesh(core_axis_name="c", subcore_axis_name="s")
    k = pl.kernel(
        _gather_body,
        mesh=mesh,
        out_type=[
            jax.ShapeDtypeStruct((_GB, _D), jnp.float32),
            jax.ShapeDtypeStruct((_GB, _D), jnp.float32),
        ],
        scratch_types=[
            pltpu.VMEM((_BPW,), jnp.int32),
            pltpu.VMEM((_BPW, _D), jnp.float32),
            pltpu.VMEM((_BPW, _D), jnp.float32),
            pltpu.SemaphoreType.DMA,
            pltpu.SemaphoreType.DMA,
        ],
    )
    return k(vid_feats, reg_feats, gidx)


def kernel(raw_preds, vid_features, reg_features):
    raw4 = raw_preds.reshape(_B, 4 + _NCLS, _R, _C)
    gidx_out, box_out, score_out = pl.pallas_call(
        _select_body,
        out_shape=[
            jax.ShapeDtypeStruct((_B, _SLOTS), jnp.int32),
            jax.ShapeDtypeStruct((_B, 4, _SLOTS), jnp.float32),
            jax.ShapeDtypeStruct((_B, _SLOTS), jnp.float32),
        ],
    )(raw4)

    boxes = jnp.transpose(box_out[:, :, :_KEEP], (0, 2, 1)).reshape(1, _B * _KEEP, 4)
    scores = score_out[:, :_KEEP].reshape(1, _B * _KEEP)

    vrows, rrows = _sc_gather(vid_features, reg_features, gidx_out.reshape(_GB))
    concat_cls = vrows.reshape(_B, _SLOTS, _D)[:, :_KEEP].reshape(1, _B * _KEEP, _D)
    concat_reg = rrows.reshape(_B, _SLOTS, _D)[:, :_KEEP].reshape(1, _B * _KEEP, _D)
    return concat_cls, concat_reg, boxes, scores


# speedup vs baseline: 26.2038x; 1.1667x over previous
import jax
import jax.numpy as jnp
from jax import lax
from jax.experimental import pallas as pl
from jax.experimental.pallas import tpu as pltpu
from jax.experimental.pallas import tpu_sc as plsc

_B = 4
_A = 20000
_R, _C = 8, 2560
_K = 750
_KEEP = 30
_NCLS = 30
_IOU = 0.75
_SLOTS = 64
_D = 256

_GB = _B * _SLOTS
_NW = 32
_BPW = _GB // _NW


def _red_max(x):
    return jnp.max(x, axis=(1, 2))


def _red_min(x):
    return jnp.min(x, axis=(1, 2))


def _red_sum_bool(x):
    return jnp.sum(x.astype(jnp.int32), axis=(1, 2))


def _compute(scr_ref, gidx_ref, box_ref, score_ref):
    x1 = scr_ref[:, 0]
    y1 = scr_ref[:, 1]
    x2 = scr_ref[:, 2]
    y2 = scr_ref[:, 3]

    s = scr_ref[:, 4]
    c = jnp.zeros_like(s)
    for j in range(1, _NCLS):
        v = scr_ref[:, 4 + j]
        c = jnp.where(v > s, jnp.float32(j), c)
        s = jnp.maximum(s, v)

    ridx = lax.broadcasted_iota(jnp.int32, (_B, _R, _C), 1)
    cidx = lax.broadcasted_iota(jnp.int32, (_B, _R, _C), 2)
    idx = ridx * _C + cidx
    in_bounds = idx < _A

    bits = jnp.where(in_bounds, lax.bitcast_convert_type(s, jnp.int32), -1)

    kk = jnp.int32(_K)

    def val_probe(lohi, nprobe):
        lo, hi = lohi
        step = (hi - lo) >> 3 if nprobe == 7 else (hi - lo) >> 1
        ms = [lo + step * (j + 1) for j in range(nprobe)]
        ges = [_red_sum_bool(bits > m[:, None, None]) >= kk for m in ms]
        for j in range(nprobe):
            lo = jnp.where(ges[j], ms[j], lo)
        for j in reversed(range(nprobe)):
            hi = jnp.where(ges[j], hi, ms[j])
        return lo, hi

    lohi = (jnp.full((_B,), -1, jnp.int32), jnp.full((_B,), 0x7F800000, jnp.int32))
    lohi = lax.fori_loop(0, 10, lambda _, lh: val_probe(lh, 7), lohi)
    lohi = lax.fori_loop(0, 4, lambda _, lh: val_probe(lh, 1), lohi)
    t = lohi[1]
    t3 = t[:, None, None]

    need = kk - _red_sum_bool(bits > t3)
    eq = bits == t3

    def idx_probe(lohi, nprobe):
        lo, hi = lohi
        step = (hi - lo) >> 3 if nprobe == 7 else (hi - lo) >> 1
        ms = [lo + step * (j + 1) for j in range(nprobe)]
        ges = [_red_sum_bool(eq & (idx < m[:, None, None])) >= need for m in ms]
        for j in range(nprobe):
            hi = jnp.where(ges[j], jnp.minimum(hi, ms[j]), hi)
        for j in reversed(range(nprobe)):
            lo = jnp.where(ges[j], lo, jnp.maximum(lo, ms[j]))
        return lo, hi

    lohi = (jnp.zeros((_B,), jnp.int32), jnp.full((_B,), _A, jnp.int32))
    lohi = lax.fori_loop(0, 4, lambda _, lh: idx_probe(lh, 7), lohi)
    lohi = lax.fori_loop(0, 4, lambda _, lh: idx_probe(lh, 1), lohi)
    cut = lohi[1]

    avail = (bits > t3) | (eq & (idx < cut[:, None, None]))

    neg = jnp.float32(-jnp.inf)
    bigi = jnp.int32(2**30)

    m4 = jnp.maximum(jnp.maximum(x1, y1), jnp.maximum(x2, y2))
    max_coord = _red_max(jnp.where(avail, m4, neg))
    shift = max_coord + 1.0
    off = c * shift[:, None, None]
    ox1 = x1 + off
    oy1 = y1 + off
    ox2 = x2 + off
    oy2 = y2 + off
    area = (x2 - x1) * (y2 - y1)

    def extract(vec, mask_p):
        return jnp.sum(jnp.where(mask_p, vec, 0.0), axis=(1, 2))

    s_av = jnp.where(avail, s, neg)
    top_m = _red_max(s_av)
    top_p = _red_min(jnp.where(avail & (s == top_m[:, None, None]), idx, bigi))

    slot = lax.broadcasted_iota(jnp.int32, (_B, _SLOTS), 1)
    gidx_acc = jnp.zeros((_B, _SLOTS), jnp.int32)
    sc_acc = jnp.zeros((_B, _SLOTS), jnp.float32)
    bx = [jnp.zeros((_B, _SLOTS), jnp.float32) for _ in range(4)]

    avail_cur = avail
    for i in range(_KEEP):
        sa = jnp.where(avail_cur, s, neg)
        m = _red_max(sa)
        valid = m > neg
        p0 = _red_min(jnp.where(avail_cur & (s == m[:, None, None]), idx, bigi))
        p = jnp.where(valid, p0, top_p)
        mask_p = idx == p[:, None, None]
        px1 = extract(x1, mask_p)
        py1 = extract(y1, mask_p)
        px2 = extract(x2, mask_p)
        py2 = extract(y2, mask_p)
        pc = extract(c, mask_p)
        ps = jnp.where(valid, m, top_m)
        po = pc * shift
        ix1 = jnp.maximum((px1 + po)[:, None, None], ox1)
        iy1 = jnp.maximum((py1 + po)[:, None, None], oy1)
        ix2 = jnp.minimum((px2 + po)[:, None, None], ox2)
        iy2 = jnp.minimum((py2 + po)[:, None, None], oy2)
        inter = jnp.maximum(ix2 - ix1, 0.0) * jnp.maximum(iy2 - iy1, 0.0)
        area_r = (px2 - px1) * (py2 - py1)
        iou = inter / ((area_r[:, None, None] + area - inter) + 1e-9)
        sup = ((iou > _IOU) | mask_p) & valid[:, None, None]
        avail_cur = avail_cur & jnp.logical_not(sup)
        hit = slot == i
        gidx_acc = jnp.where(hit, p[:, None], gidx_acc)
        sc_acc = jnp.where(hit, ps[:, None], sc_acc)
        bx[0] = jnp.where(hit, px1[:, None], bx[0])
        bx[1] = jnp.where(hit, py1[:, None], bx[1])
        bx[2] = jnp.where(hit, px2[:, None], bx[2])
        bx[3] = jnp.where(hit, py2[:, None], bx[3])

    gidx_ref[...] = gidx_acc
    score_ref[...] = sc_acc
    for j in range(4):
        box_ref[:, j, :] = bx[j]


def _select_body(raw_ref, gidx_ref, box_ref, score_ref, scr_ref):
    r = pl.program_id(0)
    scr_ref[:, :, r, :] = raw_ref[...]

    @pl.when(r == _R - 1)
    def _():
        _compute(scr_ref, gidx_ref, box_ref, score_ref)


def _gather_body(vid_hbm, reg_hbm, idx_hbm, out_v, out_r,
                 idx_v, rows_v, rows_r, sem_v, sem_r):
    wid = lax.axis_index("s") * 2 + lax.axis_index("c")
    base = wid * _BPW
    b = base // _SLOTS
    pltpu.sync_copy(idx_hbm.at[pl.ds(base, _BPW)], idx_v)
    cp_v = pltpu.async_copy(vid_hbm.at[b].at[idx_v], rows_v, sem_v)
    cp_r = pltpu.async_copy(reg_hbm.at[b].at[idx_v], rows_r, sem_r)
    cp_v.wait()
    cp_r.wait()
    pltpu.sync_copy(rows_v, out_v.at[pl.ds(base, _BPW)])
    pltpu.sync_copy(rows_r, out_r.at[pl.ds(base, _BPW)])


def _sc_gather(vid_feats, reg_feats, gidx):
    mesh = plsc.VectorSubcoreMesh(core_axis_name="c", subcore_axis_name="s")
    k = pl.kernel(
        _gather_body,
        mesh=mesh,
        out_type=[
            jax.ShapeDtypeStruct((_GB, _D), jnp.float32),
            jax.ShapeDtypeStruct((_GB, _D), jnp.float32),
        ],
        scratch_types=[
            pltpu.VMEM((_BPW,), jnp.int32),
            pltpu.VMEM((_BPW, _D), jnp.float32),
            pltpu.VMEM((_BPW, _D), jnp.float32),
            pltpu.SemaphoreType.DMA,
            pltpu.SemaphoreType.DMA,
        ],
    )
    return k(vid_feats, reg_feats, gidx)


def kernel(raw_preds, vid_features, reg_features):
    gidx_out, box_out, score_out = pl.pallas_call(
        _select_body,
        grid=(_R,),
        in_specs=[pl.BlockSpec((_B, 4 + _NCLS, _C), lambda r: (0, 0, r))],
        out_specs=[
            pl.BlockSpec((_B, _SLOTS), lambda r: (0, 0)),
            pl.BlockSpec((_B, 4, _SLOTS), lambda r: (0, 0, 0)),
            pl.BlockSpec((_B, _SLOTS), lambda r: (0, 0)),
        ],
        out_shape=[
            jax.ShapeDtypeStruct((_B, _SLOTS), jnp.int32),
            jax.ShapeDtypeStruct((_B, 4, _SLOTS), jnp.float32),
            jax.ShapeDtypeStruct((_B, _SLOTS), jnp.float32),
        ],
        scratch_shapes=[pltpu.VMEM((_B, 4 + _NCLS, _R, _C), jnp.float32)],
    )(raw_preds)

    boxes = jnp.transpose(box_out[:, :, :_KEEP], (0, 2, 1)).reshape(1, _B * _KEEP, 4)
    scores = score_out[:, :_KEEP].reshape(1, _B * _KEEP)

    vrows, rrows = _sc_gather(vid_features, reg_features, gidx_out.reshape(_GB))
    concat_cls = vrows.reshape(_B, _SLOTS, _D)[:, :_KEEP].reshape(1, _B * _KEEP, _D)
    concat_reg = rrows.reshape(_B, _SLOTS, _D)[:, :_KEEP].reshape(1, _B * _KEEP, _D)
    return concat_cls, concat_reg, boxes, scores


# speedup vs baseline: 30.0797x; 1.1479x over previous
import jax
import jax.numpy as jnp
from jax import lax
from jax.experimental import pallas as pl
from jax.experimental.pallas import tpu as pltpu
from jax.experimental.pallas import tpu_sc as plsc

_B = 4
_A = 20000
_R, _C = 8, 2560
_K = 750
_KEEP = 30
_NCLS = 30
_IOU = 0.75
_SLOTS = 64
_D = 256

_GB = _B * _SLOTS
_NW = 32
_BPW = _GB // _NW


def _red_max(x):
    return jnp.max(x, axis=(1, 2))


def _red_min(x):
    return jnp.min(x, axis=(1, 2))


def _red_sum_bool(x):
    return jnp.sum(x.astype(jnp.int32), axis=(1, 2))


def _compute(scr_ref, gidx_ref, box_ref, score_ref):
    x1 = scr_ref[0]
    y1 = scr_ref[1]
    x2 = scr_ref[2]
    y2 = scr_ref[3]

    s = scr_ref[4]
    c = jnp.zeros_like(s)
    for j in range(1, _NCLS):
        v = scr_ref[4 + j]
        c = jnp.where(v > s, jnp.float32(j), c)
        s = jnp.maximum(s, v)

    ridx = lax.broadcasted_iota(jnp.int32, (_B, _R, _C), 1)
    cidx = lax.broadcasted_iota(jnp.int32, (_B, _R, _C), 2)
    idx = ridx * _C + cidx
    in_bounds = idx < _A

    bits = jnp.where(in_bounds, lax.bitcast_convert_type(s, jnp.int32), -1)

    kk = jnp.int32(_K)

    def val_probe(lohi, nprobe):
        lo, hi = lohi
        step = (hi - lo) >> 3 if nprobe == 7 else (hi - lo) >> 1
        ms = [lo + step * (j + 1) for j in range(nprobe)]
        ges = [_red_sum_bool(bits > m[:, None, None]) >= kk for m in ms]
        for j in range(nprobe):
            lo = jnp.where(ges[j], ms[j], lo)
        for j in reversed(range(nprobe)):
            hi = jnp.where(ges[j], hi, ms[j])
        return lo, hi

    lohi = (jnp.full((_B,), -1, jnp.int32), jnp.full((_B,), 0x7F800000, jnp.int32))
    lohi = lax.fori_loop(0, 10, lambda _, lh: val_probe(lh, 7), lohi)
    lohi = lax.fori_loop(0, 4, lambda _, lh: val_probe(lh, 1), lohi)
    t = lohi[1]
    t3 = t[:, None, None]

    need = kk - _red_sum_bool(bits > t3)
    eq = bits == t3

    def idx_probe(lohi, nprobe):
        lo, hi = lohi
        step = (hi - lo) >> 3 if nprobe == 7 else (hi - lo) >> 1
        ms = [lo + step * (j + 1) for j in range(nprobe)]
        ges = [_red_sum_bool(eq & (idx < m[:, None, None])) >= need for m in ms]
        for j in range(nprobe):
            hi = jnp.where(ges[j], jnp.minimum(hi, ms[j]), hi)
        for j in reversed(range(nprobe)):
            lo = jnp.where(ges[j], lo, jnp.maximum(lo, ms[j]))
        return lo, hi

    lohi = (jnp.zeros((_B,), jnp.int32), jnp.full((_B,), _A, jnp.int32))
    lohi = lax.fori_loop(0, 4, lambda _, lh: idx_probe(lh, 7), lohi)
    lohi = lax.fori_loop(0, 4, lambda _, lh: idx_probe(lh, 1), lohi)
    cut = lohi[1]

    avail = (bits > t3) | (eq & (idx < cut[:, None, None]))

    neg = jnp.float32(-jnp.inf)
    bigi = jnp.int32(2**30)

    m4 = jnp.maximum(jnp.maximum(x1, y1), jnp.maximum(x2, y2))
    max_coord = _red_max(jnp.where(avail, m4, neg))
    shift = max_coord + 1.0
    off = c * shift[:, None, None]
    ox1 = x1 + off
    oy1 = y1 + off
    ox2 = x2 + off
    oy2 = y2 + off
    area = (x2 - x1) * (y2 - y1)

    def extract(vec, mask_p):
        return jnp.sum(jnp.where(mask_p, vec, 0.0), axis=(1, 2))

    s_av = jnp.where(avail, s, neg)
    top_m = _red_max(s_av)
    top_p = _red_min(jnp.where(avail & (s == top_m[:, None, None]), idx, bigi))

    slot = lax.broadcasted_iota(jnp.int32, (_B, _SLOTS), 1)
    gidx_acc = jnp.zeros((_B, _SLOTS), jnp.int32)
    sc_acc = jnp.zeros((_B, _SLOTS), jnp.float32)
    bx = [jnp.zeros((_B, _SLOTS), jnp.float32) for _ in range(4)]

    avail_cur = avail
    for i in range(_KEEP):
        sa = jnp.where(avail_cur, s, neg)
        m = _red_max(sa)
        valid = m > neg
        p0 = _red_min(jnp.where(avail_cur & (s == m[:, None, None]), idx, bigi))
        p = jnp.where(valid, p0, top_p)
        mask_p = idx == p[:, None, None]
        px1 = extract(x1, mask_p)
        py1 = extract(y1, mask_p)
        px2 = extract(x2, mask_p)
        py2 = extract(y2, mask_p)
        pc = extract(c, mask_p)
        ps = jnp.where(valid, m, top_m)
        po = pc * shift
        ix1 = jnp.maximum((px1 + po)[:, None, None], ox1)
        iy1 = jnp.maximum((py1 + po)[:, None, None], oy1)
        ix2 = jnp.minimum((px2 + po)[:, None, None], ox2)
        iy2 = jnp.minimum((py2 + po)[:, None, None], oy2)
        inter = jnp.maximum(ix2 - ix1, 0.0) * jnp.maximum(iy2 - iy1, 0.0)
        area_r = (px2 - px1) * (py2 - py1)
        iou = inter / ((area_r[:, None, None] + area - inter) + 1e-9)
        sup = ((iou > _IOU) | mask_p) & valid[:, None, None]
        avail_cur = avail_cur & jnp.logical_not(sup)
        hit = slot == i
        gidx_acc = jnp.where(hit, p[:, None], gidx_acc)
        sc_acc = jnp.where(hit, ps[:, None], sc_acc)
        bx[0] = jnp.where(hit, px1[:, None], bx[0])
        bx[1] = jnp.where(hit, py1[:, None], bx[1])
        bx[2] = jnp.where(hit, px2[:, None], bx[2])
        bx[3] = jnp.where(hit, py2[:, None], bx[3])

    gidx_ref[...] = gidx_acc
    score_ref[...] = sc_acc
    for j in range(4):
        box_ref[:, j, :] = bx[j]


def _select_body(raw_ref, gidx_ref, box_ref, score_ref, scr_ref):
    r = pl.program_id(0)
    scr_ref[:, :, r, :] = raw_ref[...]

    @pl.when(r == _R - 1)
    def _():
        _compute(scr_ref, gidx_ref, box_ref, score_ref)


def _gather_body(vid_hbm, reg_hbm, idx_hbm, out_v, out_r,
                 idx_v, rows_v, rows_r, sem_v, sem_r):
    wid = lax.axis_index("s") * 2 + lax.axis_index("c")
    base = wid * _BPW
    b = base // _SLOTS
    pltpu.sync_copy(idx_hbm.at[pl.ds(base, _BPW)], idx_v)
    cp_v = pltpu.async_copy(vid_hbm.at[b].at[idx_v], rows_v, sem_v)
    cp_r = pltpu.async_copy(reg_hbm.at[b].at[idx_v], rows_r, sem_r)
    cp_v.wait()
    cp_r.wait()
    pltpu.sync_copy(rows_v, out_v.at[pl.ds(base, _BPW)])
    pltpu.sync_copy(rows_r, out_r.at[pl.ds(base, _BPW)])


def _sc_gather(vid_feats, reg_feats, gidx):
    mesh = plsc.VectorSubcoreMesh(core_axis_name="c", subcore_axis_name="s")
    k = pl.kernel(
        _gather_body,
        mesh=mesh,
        out_type=[
            jax.ShapeDtypeStruct((_GB, _D), jnp.float32),
            jax.ShapeDtypeStruct((_GB, _D), jnp.float32),
        ],
        scratch_types=[
            pltpu.VMEM((_BPW,), jnp.int32),
            pltpu.VMEM((_BPW, _D), jnp.float32),
            pltpu.VMEM((_BPW, _D), jnp.float32),
            pltpu.SemaphoreType.DMA,
            pltpu.SemaphoreType.DMA,
        ],
    )
    return k(vid_feats, reg_feats, gidx)


def kernel(raw_preds, vid_features, reg_features):
    raw_t = jnp.transpose(raw_preds, (1, 0, 2))
    gidx_out, box_out, score_out = pl.pallas_call(
        _select_body,
        grid=(_R,),
        in_specs=[pl.BlockSpec((4 + _NCLS, _B, _C), lambda r: (0, 0, r))],
        out_specs=[
            pl.BlockSpec((_B, _SLOTS), lambda r: (0, 0)),
            pl.BlockSpec((_B, 4, _SLOTS), lambda r: (0, 0, 0)),
            pl.BlockSpec((_B, _SLOTS), lambda r: (0, 0)),
        ],
        out_shape=[
            jax.ShapeDtypeStruct((_B, _SLOTS), jnp.int32),
            jax.ShapeDtypeStruct((_B, 4, _SLOTS), jnp.float32),
            jax.ShapeDtypeStruct((_B, _SLOTS), jnp.float32),
        ],
        scratch_shapes=[pltpu.VMEM((4 + _NCLS, _B, _R, _C), jnp.float32)],
    )(raw_t)

    boxes = jnp.transpose(box_out[:, :, :_KEEP], (0, 2, 1)).reshape(1, _B * _KEEP, 4)
    scores = score_out[:, :_KEEP].reshape(1, _B * _KEEP)

    vrows, rrows = _sc_gather(vid_features, reg_features, gidx_out.reshape(_GB))
    concat_cls = vrows.reshape(_B, _SLOTS, _D)[:, :_KEEP].reshape(1, _B * _KEEP, _D)
    concat_reg = rrows.reshape(_B, _SLOTS, _D)[:, :_KEEP].reshape(1, _B * _KEEP, _D)
    return concat_cls, concat_reg, boxes, scores


# speedup vs baseline: 31.5593x; 1.0492x over previous
import jax
import jax.numpy as jnp
from jax import lax
from jax.experimental import pallas as pl
from jax.experimental.pallas import tpu as pltpu
from jax.experimental.pallas import tpu_sc as plsc

_B = 4
_A = 20000
_R, _C = 8, 2560
_K = 750
_KEEP = 30
_NCLS = 30
_IOU = 0.75
_SLOTS = 64
_D = 256

_GB = _B * _SLOTS
_NW = 32
_BPW = _GB // _NW


def _red_max(x):
    return jnp.max(x, axis=(1, 2))


def _red_min(x):
    return jnp.min(x, axis=(1, 2))


def _red_sum_bool(x):
    return jnp.sum(x.astype(jnp.int32), axis=(1, 2))


def _compute(scr_ref, gidx_ref, box_ref, score_ref):
    x1 = scr_ref[0]
    y1 = scr_ref[1]
    x2 = scr_ref[2]
    y2 = scr_ref[3]

    s = scr_ref[4]
    c = jnp.zeros_like(s)
    for j in range(1, _NCLS):
        v = scr_ref[4 + j]
        c = jnp.where(v > s, jnp.float32(j), c)
        s = jnp.maximum(s, v)

    ridx = lax.broadcasted_iota(jnp.int32, (_B, _R, _C), 1)
    cidx = lax.broadcasted_iota(jnp.int32, (_B, _R, _C), 2)
    idx = ridx * _C + cidx
    in_bounds = idx < _A

    bits = jnp.where(in_bounds, lax.bitcast_convert_type(s, jnp.int32), -1)

    kk = jnp.int32(_K)

    def val_probe(lohi, nprobe):
        lo, hi = lohi
        step = (hi - lo) >> 3 if nprobe == 7 else (hi - lo) >> 1
        ms = [lo + step * (j + 1) for j in range(nprobe)]
        ges = [_red_sum_bool(bits > m[:, None, None]) >= kk for m in ms]
        for j in range(nprobe):
            lo = jnp.where(ges[j], ms[j], lo)
        for j in reversed(range(nprobe)):
            hi = jnp.where(ges[j], hi, ms[j])
        return lo, hi

    lohi = (jnp.full((_B,), -1, jnp.int32), jnp.full((_B,), 0x7F800000, jnp.int32))
    lohi = lax.fori_loop(0, 10, lambda _, lh: val_probe(lh, 7), lohi)
    lohi = lax.fori_loop(0, 4, lambda _, lh: val_probe(lh, 1), lohi)
    t = lohi[1]
    t3 = t[:, None, None]

    need = kk - _red_sum_bool(bits > t3)
    eq = bits == t3

    def idx_probe(lohi, nprobe):
        lo, hi = lohi
        step = (hi - lo) >> 3 if nprobe == 7 else (hi - lo) >> 1
        ms = [lo + step * (j + 1) for j in range(nprobe)]
        ges = [_red_sum_bool(eq & (idx < m[:, None, None])) >= need for m in ms]
        for j in range(nprobe):
            hi = jnp.where(ges[j], jnp.minimum(hi, ms[j]), hi)
        for j in reversed(range(nprobe)):
            lo = jnp.where(ges[j], lo, jnp.maximum(lo, ms[j]))
        return lo, hi

    lohi = (jnp.zeros((_B,), jnp.int32), jnp.full((_B,), _A, jnp.int32))
    lohi = lax.fori_loop(0, 4, lambda _, lh: idx_probe(lh, 7), lohi)
    lohi = lax.fori_loop(0, 4, lambda _, lh: idx_probe(lh, 1), lohi)
    cut = lohi[1]

    avail = (bits > t3) | (eq & (idx < cut[:, None, None]))

    neg = jnp.float32(-jnp.inf)
    bigi = jnp.int32(2**30)

    m4 = jnp.maximum(jnp.maximum(x1, y1), jnp.maximum(x2, y2))
    max_coord = _red_max(jnp.where(avail, m4, neg))
    shift = max_coord + 1.0
    off = c * shift[:, None, None]
    ox1 = x1 + off
    oy1 = y1 + off
    ox2 = x2 + off
    oy2 = y2 + off
    area = (x2 - x1) * (y2 - y1)

    def extract(vec, mask_p):
        return jnp.sum(jnp.where(mask_p, vec, 0.0), axis=(1, 2))

    sa = jnp.where(avail, s, neg)

    top_m = _red_max(sa)
    top_p = _red_min(jnp.where(sa == top_m[:, None, None], idx, bigi))

    slot = lax.broadcasted_iota(jnp.int32, (_B, _SLOTS), 1)
    gidx_acc = jnp.zeros((_B, _SLOTS), jnp.int32)
    sc_acc = jnp.zeros((_B, _SLOTS), jnp.float32)
    bx = [jnp.zeros((_B, _SLOTS), jnp.float32) for _ in range(4)]

    for i in range(_KEEP):
        m = _red_max(sa)
        valid = m > neg
        p0 = _red_min(jnp.where(sa == m[:, None, None], idx, bigi))
        p = jnp.where(valid, p0, top_p)
        mask_p = idx == p[:, None, None]
        px1 = extract(x1, mask_p)
        py1 = extract(y1, mask_p)
        px2 = extract(x2, mask_p)
        py2 = extract(y2, mask_p)
        pc = extract(c, mask_p)
        ps = jnp.where(valid, m, top_m)
        po = pc * shift
        ix1 = jnp.maximum((px1 + po)[:, None, None], ox1)
        iy1 = jnp.maximum((py1 + po)[:, None, None], oy1)
        ix2 = jnp.minimum((px2 + po)[:, None, None], ox2)
        iy2 = jnp.minimum((py2 + po)[:, None, None], oy2)
        inter = jnp.maximum(ix2 - ix1, 0.0) * jnp.maximum(iy2 - iy1, 0.0)
        area_r = (px2 - px1) * (py2 - py1)
        iou = inter / ((area_r[:, None, None] + area - inter) + 1e-9)
        sup = ((iou > _IOU) | mask_p) & valid[:, None, None]
        sa = jnp.where(sup, neg, sa)
        hit = slot == i
        gidx_acc = jnp.where(hit, p[:, None], gidx_acc)
        sc_acc = jnp.where(hit, ps[:, None], sc_acc)
        bx[0] = jnp.where(hit, px1[:, None], bx[0])
        bx[1] = jnp.where(hit, py1[:, None], bx[1])
        bx[2] = jnp.where(hit, px2[:, None], bx[2])
        bx[3] = jnp.where(hit, py2[:, None], bx[3])

    gidx_ref[...] = gidx_acc
    score_ref[...] = sc_acc
    for j in range(4):
        box_ref[:, j, :] = bx[j]


def _select_body(raw_ref, gidx_ref, box_ref, score_ref, scr_ref):
    r = pl.program_id(0)
    scr_ref[:, :, r, :] = raw_ref[...]

    @pl.when(r == _R - 1)
    def _():
        _compute(scr_ref, gidx_ref, box_ref, score_ref)


def _gather_body(vid_hbm, reg_hbm, idx_hbm, out_v, out_r,
                 idx_v, rows_v, rows_r, sem_v, sem_r):
    wid = lax.axis_index("s") * 2 + lax.axis_index("c")
    base = wid * _BPW
    b = base // _SLOTS
    pltpu.sync_copy(idx_hbm.at[pl.ds(base, _BPW)], idx_v)
    cp_v = pltpu.async_copy(vid_hbm.at[b].at[idx_v], rows_v, sem_v)
    cp_r = pltpu.async_copy(reg_hbm.at[b].at[idx_v], rows_r, sem_r)
    cp_v.wait()
    cp_r.wait()
    pltpu.sync_copy(rows_v, out_v.at[pl.ds(base, _BPW)])
    pltpu.sync_copy(rows_r, out_r.at[pl.ds(base, _BPW)])


def _sc_gather(vid_feats, reg_feats, gidx):
    mesh = plsc.VectorSubcoreMesh(core_axis_name="c", subcore_axis_name="s")
    k = pl.kernel(
        _gather_body,
        mesh=mesh,
        out_type=[
            jax.ShapeDtypeStruct((_GB, _D), jnp.float32),
            jax.ShapeDtypeStruct((_GB, _D), jnp.float32),
        ],
        scratch_types=[
            pltpu.VMEM((_BPW,), jnp.int32),
            pltpu.VMEM((_BPW, _D), jnp.float32),
            pltpu.VMEM((_BPW, _D), jnp.float32),
            pltpu.SemaphoreType.DMA,
            pltpu.SemaphoreType.DMA,
        ],
    )
    return k(vid_feats, reg_feats, gidx)


def kernel(raw_preds, vid_features, reg_features):
    raw_t = jnp.transpose(raw_preds, (1, 0, 2))
    gidx_out, box_out, score_out = pl.pallas_call(
        _select_body,
        grid=(_R,),
        in_specs=[pl.BlockSpec((4 + _NCLS, _B, _C), lambda r: (0, 0, r))],
        out_specs=[
            pl.BlockSpec((_B, _SLOTS), lambda r: (0, 0)),
            pl.BlockSpec((_B, 4, _SLOTS), lambda r: (0, 0, 0)),
            pl.BlockSpec((_B, _SLOTS), lambda r: (0, 0)),
        ],
        out_shape=[
            jax.ShapeDtypeStruct((_B, _SLOTS), jnp.int32),
            jax.ShapeDtypeStruct((_B, 4, _SLOTS), jnp.float32),
            jax.ShapeDtypeStruct((_B, _SLOTS), jnp.float32),
        ],
        scratch_shapes=[pltpu.VMEM((4 + _NCLS, _B, _R, _C), jnp.float32)],
    )(raw_t)

    boxes = jnp.transpose(box_out[:, :, :_KEEP], (0, 2, 1)).reshape(1, _B * _KEEP, 4)
    scores = score_out[:, :_KEEP].reshape(1, _B * _KEEP)

    vrows, rrows = _sc_gather(vid_features, reg_features, gidx_out.reshape(_GB))
    concat_cls = vrows.reshape(_B, _SLOTS, _D)[:, :_KEEP].reshape(1, _B * _KEEP, _D)
    concat_reg = rrows.reshape(_B, _SLOTS, _D)[:, :_KEEP].reshape(1, _B * _KEEP, _D)
    return concat_cls, concat_reg, boxes, scores
